# initial kernel scaffold (unmeasured)
import jax
import jax.numpy as jnp
from jax import lax
from jax.experimental import pallas as pl
from jax.experimental.pallas import tpu as pltpu


def kernel(Q, K, V):
    b, sq, h, d = Q.shape
    skv = K.shape[1]
    scale = d ** -0.5

    def attn_body(q_ref, k_ref, v_ref, o_ref, l_ref):
        q = q_ref[0, :, 0, :]
        k = k_ref[0, :, 0, :]
        v = v_ref[0, :, 0, :]
        s = lax.dot_general(
            q, k, (((1,), (1,)), ((), ())),
            preferred_element_type=jnp.float32,
        ) * scale
        p = jnp.exp(s)
        l_ref[0, :, 0] = jnp.sum(p, axis=1)
        o_ref[0, :, 0, :] = lax.dot_general(
            p, v, (((1,), (0,)), ((), ())),
            preferred_element_type=jnp.float32,
        )

    o_part, l_part = pl.pallas_call(
        attn_body,
        grid=(b, h),
        in_specs=[
            pl.BlockSpec((1, sq, 1, d), lambda bi, hi: (bi, 0, hi, 0)),
            pl.BlockSpec((1, skv, 1, d), lambda bi, hi: (bi, 0, hi, 0)),
            pl.BlockSpec((1, skv, 1, d), lambda bi, hi: (bi, 0, hi, 0)),
        ],
        out_specs=[
            pl.BlockSpec((1, sq, 1, d), lambda bi, hi: (bi, 0, hi, 0)),
            pl.BlockSpec((1, sq, 1), lambda bi, hi: (bi, 0, hi)),
        ],
        out_shape=[
            jax.ShapeDtypeStruct((b, sq, h, d), jnp.float32),
            jax.ShapeDtypeStruct((b, sq, h), jnp.float32),
        ],
    )(Q, K, V)

    def combine_body(o_ref, l_ref, out_ref, o_rx, l_rx, send_sems, recv_sems):
        my_x = lax.axis_index("x")
        my_y = lax.axis_index("y")
        nbr = (1 - my_x, my_y)

        bsem = pltpu.get_barrier_semaphore()
        pl.semaphore_signal(
            bsem, inc=1, device_id=nbr, device_id_type=pl.DeviceIdType.MESH
        )
        pl.semaphore_wait(bsem, 1)

        rdma_o = pltpu.make_async_remote_copy(
            src_ref=o_ref, dst_ref=o_rx,
            send_sem=send_sems.at[0], recv_sem=recv_sems.at[0],
            device_id=nbr, device_id_type=pl.DeviceIdType.MESH,
        )
        rdma_l = pltpu.make_async_remote_copy(
            src_ref=l_ref, dst_ref=l_rx,
            send_sem=send_sems.at[1], recv_sem=recv_sems.at[1],
            device_id=nbr, device_id_type=pl.DeviceIdType.MESH,
        )
        rdma_o.start()
        rdma_l.start()
        rdma_o.wait()
        rdma_l.wait()

        denom = l_ref[...] + l_rx[...]
        out_ref[...] = (o_ref[...] + o_rx[...]) / denom[..., None]

    return pl.pallas_call(
        combine_body,
        in_specs=[
            pl.BlockSpec(memory_space=pltpu.VMEM),
            pl.BlockSpec(memory_space=pltpu.VMEM),
        ],
        out_specs=pl.BlockSpec(memory_space=pltpu.VMEM),
        out_shape=jax.ShapeDtypeStruct((b, sq, h, d), jnp.float32),
        scratch_shapes=[
            pltpu.VMEM((b, sq, h, d), jnp.float32),
            pltpu.VMEM((b, sq, h), jnp.float32),
            pltpu.SemaphoreType.DMA((2,)),
            pltpu.SemaphoreType.DMA((2,)),
        ],
        compiler_params=pltpu.CompilerParams(collective_id=0),
    )(o_part, l_part)


# baseline (device time: 92403 ns/iter reference)
import jax
import jax.numpy as jnp
from jax import lax
from jax.experimental import pallas as pl
from jax.experimental.pallas import tpu as pltpu


def kernel(Q, K, V):
    b, sq, h, d = Q.shape
    skv = K.shape[1]
    scale = d ** -0.5

    nc = 4
    ck = skv // nc

    def attn_body(q_ref, k_ref, v_ref, o_ref, l_ref):
        c = pl.program_id(1)

        @pl.when(c == 0)
        def _():
            o_ref[...] = jnp.zeros_like(o_ref)
            l_ref[...] = jnp.zeros_like(l_ref)

        for hi in range(h):
            q = q_ref[0, :, hi, :]
            k = k_ref[0, :, hi, :]
            v = v_ref[0, :, hi, :]
            s = lax.dot_general(
                q, k, (((1,), (1,)), ((), ())),
                preferred_element_type=jnp.float32,
            ) * scale
            p = jnp.exp(s)
            l_ref[0, :, hi] += jnp.sum(p, axis=1)
            o_ref[0, :, hi, :] += lax.dot_general(
                p, v, (((1,), (0,)), ((), ())),
                preferred_element_type=jnp.float32,
            )

    o_part, l_part = pl.pallas_call(
        attn_body,
        grid=(b, nc),
        in_specs=[
            pl.BlockSpec((1, sq, h, d), lambda bi, ci: (bi, 0, 0, 0)),
            pl.BlockSpec((1, ck, h, d), lambda bi, ci: (bi, ci, 0, 0)),
            pl.BlockSpec((1, ck, h, d), lambda bi, ci: (bi, ci, 0, 0)),
        ],
        out_specs=[
            pl.BlockSpec((1, sq, h, d), lambda bi, ci: (bi, 0, 0, 0)),
            pl.BlockSpec((1, sq, h), lambda bi, ci: (bi, 0, 0)),
        ],
        out_shape=[
            jax.ShapeDtypeStruct((b, sq, h, d), jnp.float32),
            jax.ShapeDtypeStruct((b, sq, h), jnp.float32),
        ],
    )(Q, K, V)

    def combine_body(o_ref, l_ref, out_ref, o_rx, l_rx, send_sems, recv_sems):
        my_x = lax.axis_index("x")
        my_y = lax.axis_index("y")
        nbr = (1 - my_x, my_y)

        bsem = pltpu.get_barrier_semaphore()
        pl.semaphore_signal(
            bsem, inc=1, device_id=nbr, device_id_type=pl.DeviceIdType.MESH
        )
        pl.semaphore_wait(bsem, 1)

        rdma_o = pltpu.make_async_remote_copy(
            src_ref=o_ref, dst_ref=o_rx,
            send_sem=send_sems.at[0], recv_sem=recv_sems.at[0],
            device_id=nbr, device_id_type=pl.DeviceIdType.MESH,
        )
        rdma_l = pltpu.make_async_remote_copy(
            src_ref=l_ref, dst_ref=l_rx,
            send_sem=send_sems.at[1], recv_sem=recv_sems.at[1],
            device_id=nbr, device_id_type=pl.DeviceIdType.MESH,
        )
        rdma_o.start()
        rdma_l.start()
        rdma_o.wait()
        rdma_l.wait()

        denom = l_ref[...] + l_rx[...]
        out_ref[...] = (o_ref[...] + o_rx[...]) / denom[..., None]

    return pl.pallas_call(
        combine_body,
        in_specs=[
            pl.BlockSpec(memory_space=pltpu.VMEM),
            pl.BlockSpec(memory_space=pltpu.VMEM),
        ],
        out_specs=pl.BlockSpec(memory_space=pltpu.VMEM),
        out_shape=jax.ShapeDtypeStruct((b, sq, h, d), jnp.float32),
        scratch_shapes=[
            pltpu.VMEM((b, sq, h, d), jnp.float32),
            pltpu.VMEM((b, sq, h), jnp.float32),
            pltpu.SemaphoreType.DMA((2,)),
            pltpu.SemaphoreType.DMA((2,)),
        ],
        compiler_params=pltpu.CompilerParams(collective_id=0),
    )(o_part, l_part)
